# baseline (speedup 1.0000x reference)
import functools

import jax
import jax.numpy as jnp
from jax import lax
from jax.experimental import pallas as pl
from jax.experimental.pallas import tpu as pltpu
from jax.experimental.pallas import tpu_sc as plsc

B = 256
T = 16
F = 16080
C = 1000
CP = 1024
K = 200

_NV = 63

ZF = 1024
NZQ = 8
NHQ = 2

_mesh = plsc.VectorSubcoreMesh(core_axis_name="c", subcore_axis_name="s")


@functools.partial(
    pl.kernel,
    out_type=jax.ShapeDtypeStruct((T, CP), jnp.float32),
    mesh=_mesh,
    scratch_types=[
        pltpu.VMEM((_NV * 16,), jnp.float32),
        pltpu.VMEM((CP,), jnp.float32),
    ],
)
def _sc_topk(mask_hbm, head_hbm, sbuf, hbuf):
    c = lax.axis_index("c")
    s = lax.axis_index("s")

    @pl.when(c == 0)
    def _():
        t = s
        pltpu.sync_copy(mask_hbm.at[pl.ds(t * C, C)], sbuf.at[pl.ds(0, C)])

        lanes = lax.iota(jnp.int32, 16)
        for j in range(_NV):
            m = sbuf[pl.ds(j * 16, 16)]
            sv = 1.0 / (1.0 + jnp.exp(-m))
            if (j + 1) * 16 > C:
                sv = jnp.where(lanes + j * 16 < C, sv, 0.0)
            sbuf[pl.ds(j * 16, 16)] = sv

        def count_if(pred):
            acc = jnp.zeros((16,), jnp.int32)
            for j in range(_NV):
                sv = sbuf[pl.ds(j * 16, 16)]
                colv = lanes + j * 16
                acc = acc + jnp.where(pred(sv, colv), 1, 0)
            tot = acc[0]
            for l in range(1, 16):
                tot = tot + acc[l]
            return tot

        def splat(v):
            return jnp.full((16,), v)

        def bstep(_, lohi):
            lo, hi = lohi
            mid = (lo + hi) * jnp.float32(0.5)
            ge = count_if(lambda sv, colv: sv >= splat(mid)) >= K
            return (jnp.where(ge, mid, lo), jnp.where(ge, hi, mid))

        lo, hi = lax.fori_loop(
            0, 50, bstep, (jnp.float32(0.0), jnp.float32(1.0)))
        los, his = splat(lo), splat(hi)

        need = K - count_if(lambda sv, colv: sv >= his)

        def istep(_, lohi):
            ilo, ihi = lohi
            mid = (ilo + ihi) // 2
            ok = count_if(
                lambda sv, colv: (sv >= los) & (sv < his)
                & (colv < splat(mid))) >= need
            return (jnp.where(ok, ilo, mid + 1), jnp.where(ok, mid, ihi))

        pcut, _u2 = lax.fori_loop(
            0, 10, istep, (jnp.int32(0), jnp.int32(C)))
        pcs = splat(pcut)

        for j in range(CP // 16):
            if j < _NV:
                sv = sbuf[pl.ds(j * 16, 16)]
                colv = lanes + j * 16
                keep = (sv >= his) | ((sv >= los) & (sv < his) & (colv < pcs))
                hbuf[pl.ds(j * 16, 16)] = jnp.where(keep, sv, 0.0)
            else:
                hbuf[pl.ds(j * 16, 16)] = jnp.zeros((16,), jnp.float32)

        pltpu.sync_copy(hbuf, head_hbm.at[t])


def _tc_body(head_ref, xt_ref, out_ref, att_ref, zbuf, hbufs, zsems, hsems):
    attp = head_ref[...]
    att_ref[:, :CP] = attp
    att_ref[:, CP:] = jnp.zeros((T, F - CP), jnp.float32)

    zbuf[...] = jnp.zeros((ZF, B), jnp.float32)

    zq = 0
    zwaits = []
    for t in range(T):
        f0 = CP
        while f0 < F:
            n = min(ZF, F - f0)
            cp = pltpu.make_async_copy(
                zbuf.at[pl.ds(0, n), :],
                out_ref.at[t, pl.ds(f0, n), :],
                zsems.at[zq % NZQ],
            )
            if len(zwaits) >= NZQ:
                zwaits.pop(0).wait()
            cp.start()
            zwaits.append(cp)
            f0 += n
            zq += 1

    hprev = []
    for t in range(T):
        buf = hbufs[t % NHQ]
        if len(hprev) >= NHQ:
            hprev.pop(0).wait()
        buf[...] = xt_ref[...] * attp[t][:, None]
        cp = pltpu.make_async_copy(
            buf, out_ref.at[t, pl.ds(0, CP), :], hsems.at[t % NHQ])
        cp.start()
        hprev.append(cp)

    for cp in zwaits:
        cp.wait()
    for cp in hprev:
        cp.wait()


def kernel(x, attention_mask):
    att_head = _sc_topk(attention_mask.reshape(-1))
    xt = x[:, :6, :].reshape(B, 6 * 201)[:, :CP].T
    out_t, att = pl.pallas_call(
        _tc_body,
        in_specs=[
            pl.BlockSpec(memory_space=pltpu.VMEM),
            pl.BlockSpec(memory_space=pltpu.VMEM),
        ],
        out_specs=[
            pl.BlockSpec(memory_space=pl.MemorySpace.ANY),
            pl.BlockSpec(memory_space=pltpu.VMEM),
        ],
        out_shape=[
            jax.ShapeDtypeStruct((T, F, B), jnp.float32),
            jax.ShapeDtypeStruct((T, F), jnp.float32),
        ],
        scratch_shapes=[
            pltpu.VMEM((ZF, B), jnp.float32),
            [pltpu.VMEM((CP, B), jnp.float32) for _ in range(NHQ)],
            pltpu.SemaphoreType.DMA((NZQ,)),
            pltpu.SemaphoreType.DMA((NHQ,)),
        ],
    )(att_head, xt)
    return jnp.transpose(out_t, (2, 0, 1)), att


# speedup vs baseline: 1.0858x; 1.0858x over previous
import functools

import jax
import jax.numpy as jnp
from jax import lax
from jax.experimental import pallas as pl
from jax.experimental.pallas import tpu as pltpu
from jax.experimental.pallas import tpu_sc as plsc

B = 256
T = 16
F = 16080
C = 1000
CP = 1024
K = 200

_NV = 63

ZF = 1024
NZQ = 8
NHQ = 2

_mesh = plsc.VectorSubcoreMesh(core_axis_name="c", subcore_axis_name="s")


@functools.partial(
    pl.kernel,
    out_type=jax.ShapeDtypeStruct((T, CP), jnp.float32),
    mesh=_mesh,
    scratch_types=[
        pltpu.VMEM((_NV * 16,), jnp.float32),
        pltpu.VMEM((CP,), jnp.float32),
    ],
)
def _sc_topk(mask_hbm, head_hbm, sbuf, hbuf):
    c = lax.axis_index("c")
    s = lax.axis_index("s")

    @pl.when(c == 0)
    def _():
        t = s
        pltpu.sync_copy(mask_hbm.at[pl.ds(t * C, C)], sbuf.at[pl.ds(0, C)])

        lanes = lax.iota(jnp.int32, 16)
        for j in range(_NV):
            m = sbuf[pl.ds(j * 16, 16)]
            sv = 1.0 / (1.0 + jnp.exp(-m))
            if (j + 1) * 16 > C:
                sv = jnp.where(lanes + j * 16 < C, sv, 0.0)
            sbuf[pl.ds(j * 16, 16)] = sv

        def count_if(pred):
            acc = jnp.zeros((16,), jnp.int32)
            for j in range(_NV):
                sv = sbuf[pl.ds(j * 16, 16)]
                colv = lanes + j * 16
                acc = acc + jnp.where(pred(sv, colv), 1, 0)
            tot = acc[0]
            for l in range(1, 16):
                tot = tot + acc[l]
            return tot

        def splat(v):
            return jnp.full((16,), v)

        def bstep(_, lohi):
            lo, hi = lohi
            mid = (lo + hi) * jnp.float32(0.5)
            ge = count_if(lambda sv, colv: sv >= splat(mid)) >= K
            return (jnp.where(ge, mid, lo), jnp.where(ge, hi, mid))

        lo, hi = lax.fori_loop(
            0, 50, bstep, (jnp.float32(0.0), jnp.float32(1.0)))
        los, his = splat(lo), splat(hi)

        need = K - count_if(lambda sv, colv: sv >= his)

        def istep(_, lohi):
            ilo, ihi = lohi
            mid = (ilo + ihi) // 2
            ok = count_if(
                lambda sv, colv: (sv >= los) & (sv < his)
                & (colv < splat(mid))) >= need
            return (jnp.where(ok, ilo, mid + 1), jnp.where(ok, mid, ihi))

        pcut, _u2 = lax.fori_loop(
            0, 10, istep, (jnp.int32(0), jnp.int32(C)))
        pcs = splat(pcut)

        for j in range(CP // 16):
            if j < _NV:
                sv = sbuf[pl.ds(j * 16, 16)]
                colv = lanes + j * 16
                keep = (sv >= his) | ((sv >= los) & (sv < his) & (colv < pcs))
                hbuf[pl.ds(j * 16, 16)] = jnp.where(keep, sv, 0.0)
            else:
                hbuf[pl.ds(j * 16, 16)] = jnp.zeros((16,), jnp.float32)

        pltpu.sync_copy(hbuf, head_hbm.at[t])


def _tc_zero_body(out_ref, zbuf, zsems):
    zbuf[...] = jnp.zeros((ZF, B), jnp.float32)

    zq = 0
    zwaits = []
    for t in range(T):
        f0 = CP
        while f0 < F:
            n = min(ZF, F - f0)
            cp = pltpu.make_async_copy(
                zbuf.at[pl.ds(0, n), :],
                out_ref.at[t, pl.ds(f0, n), :],
                zsems.at[zq % NZQ],
            )
            if len(zwaits) >= NZQ:
                zwaits.pop(0).wait()
            cp.start()
            zwaits.append(cp)
            f0 += n
            zq += 1
    for cp in zwaits:
        cp.wait()


def _tc_head_body(outin_ref, head_ref, xt_ref, out_ref, att_ref,
                  hbufs, hsems):
    del outin_ref
    attp = head_ref[...]
    att_ref[:, :CP] = attp
    att_ref[:, CP:] = jnp.zeros((T, F - CP), jnp.float32)

    hprev = []
    for t in range(T):
        buf = hbufs[t % NHQ]
        if len(hprev) >= NHQ:
            hprev.pop(0).wait()
        buf[...] = xt_ref[...] * attp[t][:, None]
        cp = pltpu.make_async_copy(
            buf, out_ref.at[t, pl.ds(0, CP), :], hsems.at[t % NHQ])
        cp.start()
        hprev.append(cp)
    for cp in hprev:
        cp.wait()


def kernel(x, attention_mask):
    att_head = _sc_topk(attention_mask.reshape(-1))
    xt = x[:, :6, :].reshape(B, 6 * 201)[:, :CP].T

    out_z = pl.pallas_call(
        _tc_zero_body,
        out_specs=pl.BlockSpec(memory_space=pl.MemorySpace.ANY),
        out_shape=jax.ShapeDtypeStruct((T, F, B), jnp.float32),
        scratch_shapes=[
            pltpu.VMEM((ZF, B), jnp.float32),
            pltpu.SemaphoreType.DMA((NZQ,)),
        ],
    )()

    out_t, att = pl.pallas_call(
        _tc_head_body,
        in_specs=[
            pl.BlockSpec(memory_space=pl.MemorySpace.ANY),
            pl.BlockSpec(memory_space=pltpu.VMEM),
            pl.BlockSpec(memory_space=pltpu.VMEM),
        ],
        out_specs=[
            pl.BlockSpec(memory_space=pl.MemorySpace.ANY),
            pl.BlockSpec(memory_space=pltpu.VMEM),
        ],
        out_shape=[
            jax.ShapeDtypeStruct((T, F, B), jnp.float32),
            jax.ShapeDtypeStruct((T, F), jnp.float32),
        ],
        scratch_shapes=[
            [pltpu.VMEM((CP, B), jnp.float32) for _ in range(NHQ)],
            pltpu.SemaphoreType.DMA((NHQ,)),
        ],
        input_output_aliases={0: 0},
    )(out_z, att_head, xt)
    return jnp.transpose(out_t, (2, 0, 1)), att
